# baseline (device time: 248473 ns/iter reference)
import jax
import jax.numpy as jnp
from jax import lax
from jax.experimental import pallas as pl
from jax.experimental.pallas import tpu as pltpu

N_DEV = 16
M = 2048
N = 2048
CH = M // N_DEV


def kernel(A, B):
    a = A.astype(jnp.bfloat16)
    b = B.astype(jnp.bfloat16)

    def body(a_ref, b_ref, out_ref, part_ref, rs_recv, send_buf,
             rs_send_sems, rs_recv_sems, ag_send_sems, ag_recv_sems):
        my = lax.axis_index("i")
        left = lax.rem(my + N_DEV - 1, N_DEV)
        right = lax.rem(my + 1, N_DEV)

        barrier_sem = pltpu.get_barrier_semaphore()
        pl.semaphore_signal(barrier_sem, inc=1, device_id=(left,),
                            device_id_type=pl.DeviceIdType.MESH)
        pl.semaphore_signal(barrier_sem, inc=1, device_id=(right,),
                            device_id_type=pl.DeviceIdType.MESH)
        pl.semaphore_wait(barrier_sem, 2)

        for c in range(N_DEV):
            part_ref[c] = jnp.dot(
                a_ref[pl.ds(c * CH, CH), :], b_ref[...],
                preferred_element_type=jnp.float32,
            ).astype(jnp.bfloat16)

        def part_chunk(idx):
            return part_ref[idx].astype(jnp.float32)

        send_buf[0] = part_ref[my]
        for h in range(N_DEV - 1):
            slot = h % 2
            rdma = pltpu.make_async_remote_copy(
                src_ref=send_buf.at[slot],
                dst_ref=rs_recv.at[h],
                send_sem=rs_send_sems.at[h],
                recv_sem=rs_recv_sems.at[h],
                device_id=(right,),
                device_id_type=pl.DeviceIdType.MESH,
            )
            rdma.start()
            rdma.wait()
            recv_idx = lax.rem(my + 2 * N_DEV - 1 - h, N_DEV)
            acc = rs_recv[h].astype(jnp.float32) + part_chunk(recv_idx)
            if h < N_DEV - 2:
                send_buf[1 - slot] = acc.astype(jnp.bfloat16)
            else:
                owned = lax.rem(my + 1, N_DEV)
                out_ref[owned] = jnp.maximum(acc, 0.0).astype(jnp.bfloat16)

        for h in range(N_DEV - 1):
            g = lax.rem(my + 2 * N_DEV + 1 - h, N_DEV)
            rdma = pltpu.make_async_remote_copy(
                src_ref=out_ref.at[g],
                dst_ref=out_ref.at[g],
                send_sem=ag_send_sems.at[h],
                recv_sem=ag_recv_sems.at[h],
                device_id=(right,),
                device_id_type=pl.DeviceIdType.MESH,
            )
            rdma.start()
            rdma.wait()

    out3 = pl.pallas_call(
        body,
        out_shape=jax.ShapeDtypeStruct((N_DEV, CH, N), jnp.bfloat16),
        in_specs=[
            pl.BlockSpec(memory_space=pltpu.VMEM),
            pl.BlockSpec(memory_space=pltpu.VMEM),
        ],
        out_specs=pl.BlockSpec(memory_space=pltpu.VMEM),
        scratch_shapes=[
            pltpu.VMEM((N_DEV, CH, N), jnp.bfloat16),
            pltpu.VMEM((N_DEV - 1, CH, N), jnp.bfloat16),
            pltpu.VMEM((2, CH, N), jnp.bfloat16),
            pltpu.SemaphoreType.DMA((N_DEV - 1,)),
            pltpu.SemaphoreType.DMA((N_DEV - 1,)),
            pltpu.SemaphoreType.DMA((N_DEV - 1,)),
            pltpu.SemaphoreType.DMA((N_DEV - 1,)),
        ],
        compiler_params=pltpu.CompilerParams(collective_id=0),
    )(a, b)
    return out3.reshape(M, N)


# device time: 195642 ns/iter; 1.2700x vs baseline; 1.2700x over previous
import jax
import jax.numpy as jnp
from jax import lax
from jax.experimental import pallas as pl
from jax.experimental.pallas import tpu as pltpu

N_DEV = 16
M = 2048
N = 2048
K = 1024
CH = M // N_DEV
HN = N // 2


def kernel(A, B):
    a = A.astype(jnp.bfloat16)
    b = B.astype(jnp.bfloat16)

    def body(a_ref, b_ref, out_ref, part_ref,
             rcw, rccw, scw, sccw,
             cw_ssem, cw_rsem, ccw_ssem, ccw_rsem,
             agcw_ssem, agcw_rsem, agccw_ssem, agccw_rsem):
        my = lax.axis_index("i")
        left = lax.rem(my + N_DEV - 1, N_DEV)
        right = lax.rem(my + 1, N_DEV)

        def pidx(off):
            return lax.rem(my + 4 * N_DEV + off, N_DEV)

        barrier_sem = pltpu.get_barrier_semaphore()
        for nbr in (left, right):
            pl.semaphore_signal(barrier_sem, inc=1, device_id=(nbr,),
                                device_id_type=pl.DeviceIdType.MESH)
        pl.semaphore_wait(barrier_sem, 2)

        def compute_chunk(idx):
            part_ref[idx] = jnp.dot(
                a_ref[pl.ds(idx * CH, CH), :], b_ref[...],
                preferred_element_type=jnp.float32,
            ).astype(jnp.bfloat16)

        compute_chunk(my)
        seed = part_ref[my]
        scw[0] = seed[:, :HN]
        sccw[0] = seed[:, HN:]

        prev_cw = prev_ccw = None
        for h in range(N_DEV - 1):
            slot = h % 2
            r_cw = pltpu.make_async_remote_copy(
                src_ref=scw.at[slot], dst_ref=rcw.at[h],
                send_sem=cw_ssem.at[h], recv_sem=cw_rsem.at[h],
                device_id=(right,), device_id_type=pl.DeviceIdType.MESH,
            )
            r_ccw = pltpu.make_async_remote_copy(
                src_ref=sccw.at[slot], dst_ref=rccw.at[h],
                send_sem=ccw_ssem.at[h], recv_sem=ccw_rsem.at[h],
                device_id=(left,), device_id_type=pl.DeviceIdType.MESH,
            )
            r_cw.start()
            r_ccw.start()
            if prev_cw is not None:
                prev_cw.wait_send()
                prev_ccw.wait_send()
            if h == 0:
                compute_chunk(pidx(-1))
                compute_chunk(pidx(1))
            k = h + 2
            if k <= N_DEV // 2:
                compute_chunk(pidx(-k))
                if k < N_DEV // 2:
                    compute_chunk(pidx(k))
            r_cw.wait_recv()
            r_ccw.wait_recv()
            add_cw = part_ref[pidx(-1 - h)]
            add_ccw = part_ref[pidx(1 + h)]
            acc_cw = rcw[h].astype(jnp.float32) + add_cw[:, :HN].astype(jnp.float32)
            acc_ccw = rccw[h].astype(jnp.float32) + add_ccw[:, HN:].astype(jnp.float32)
            if h < N_DEV - 2:
                scw[1 - slot] = acc_cw.astype(jnp.bfloat16)
                sccw[1 - slot] = acc_ccw.astype(jnp.bfloat16)
            else:
                out_ref[pidx(1), :, pl.ds(0, HN)] = (
                    jnp.maximum(acc_cw, 0.0).astype(jnp.bfloat16))
                out_ref[pidx(-1), :, pl.ds(HN, HN)] = (
                    jnp.maximum(acc_ccw, 0.0).astype(jnp.bfloat16))
            prev_cw, prev_ccw = r_cw, r_ccw
        prev_cw.wait_send()
        prev_ccw.wait_send()

        prev_cw = prev_ccw = None
        for h in range(N_DEV - 1):
            gcw = pidx(1 - h)
            gccw = pidx(-1 + h)
            a_cw = pltpu.make_async_remote_copy(
                src_ref=out_ref.at[gcw, :, pl.ds(0, HN)],
                dst_ref=out_ref.at[gcw, :, pl.ds(0, HN)],
                send_sem=agcw_ssem.at[h], recv_sem=agcw_rsem.at[h],
                device_id=(right,), device_id_type=pl.DeviceIdType.MESH,
            )
            a_ccw = pltpu.make_async_remote_copy(
                src_ref=out_ref.at[gccw, :, pl.ds(HN, HN)],
                dst_ref=out_ref.at[gccw, :, pl.ds(HN, HN)],
                send_sem=agccw_ssem.at[h], recv_sem=agccw_rsem.at[h],
                device_id=(left,), device_id_type=pl.DeviceIdType.MESH,
            )
            a_cw.start()
            a_ccw.start()
            if prev_cw is not None:
                prev_cw.wait_send()
                prev_ccw.wait_send()
            a_cw.wait_recv()
            a_ccw.wait_recv()
            prev_cw, prev_ccw = a_cw, a_ccw
        prev_cw.wait_send()
        prev_ccw.wait_send()

    out3 = pl.pallas_call(
        body,
        out_shape=jax.ShapeDtypeStruct((N_DEV, CH, N), jnp.bfloat16),
        in_specs=[
            pl.BlockSpec(memory_space=pltpu.VMEM),
            pl.BlockSpec(memory_space=pltpu.VMEM),
        ],
        out_specs=pl.BlockSpec(memory_space=pltpu.VMEM),
        scratch_shapes=[
            pltpu.VMEM((N_DEV, CH, N), jnp.bfloat16),
            pltpu.VMEM((N_DEV - 1, CH, HN), jnp.bfloat16),
            pltpu.VMEM((N_DEV - 1, CH, HN), jnp.bfloat16),
            pltpu.VMEM((2, CH, HN), jnp.bfloat16),
            pltpu.VMEM((2, CH, HN), jnp.bfloat16),
            pltpu.SemaphoreType.DMA((N_DEV - 1,)),
            pltpu.SemaphoreType.DMA((N_DEV - 1,)),
            pltpu.SemaphoreType.DMA((N_DEV - 1,)),
            pltpu.SemaphoreType.DMA((N_DEV - 1,)),
            pltpu.SemaphoreType.DMA((N_DEV - 1,)),
            pltpu.SemaphoreType.DMA((N_DEV - 1,)),
            pltpu.SemaphoreType.DMA((N_DEV - 1,)),
            pltpu.SemaphoreType.DMA((N_DEV - 1,)),
        ],
        compiler_params=pltpu.CompilerParams(collective_id=0),
    )(a, b)
    return out3.reshape(M, N)


# device time: 153041 ns/iter; 1.6236x vs baseline; 1.2784x over previous
import jax
import jax.numpy as jnp
from jax import lax
from jax.experimental import pallas as pl
from jax.experimental.pallas import tpu as pltpu

N_DEV = 16
M = 2048
N = 2048
K = 1024
CH = M // N_DEV
HN = N // 2

RING = (0, 4, 8, 12, 13, 14, 15, 11, 10, 9, 5, 6, 7, 3, 2, 1)


def kernel(A, B):
    a = A.astype(jnp.bfloat16)
    b = B.astype(jnp.bfloat16)

    def body(a_ref, b_ref, out_ref, part_ref,
             rcw, rccw, scw, sccw,
             cw_ssem, cw_rsem, ccw_ssem, ccw_rsem,
             agcw_ssem, agcw_rsem, agccw_ssem, agccw_rsem):
        my = lax.axis_index("i")
        zero = my - my
        r = zero
        right = zero
        left = zero
        for j in range(N_DEV):
            sel = (my == RING[j]).astype(jnp.int32)
            r = r + j * sel
            right = right + RING[(j + 1) % N_DEV] * sel
            left = left + RING[(j - 1) % N_DEV] * sel

        def pidx(off):
            return lax.rem(r + 4 * N_DEV + off, N_DEV)

        barrier_sem = pltpu.get_barrier_semaphore()
        for nbr in (left, right):
            pl.semaphore_signal(barrier_sem, inc=1, device_id=(nbr,),
                                device_id_type=pl.DeviceIdType.MESH)
        pl.semaphore_wait(barrier_sem, 2)

        def compute_chunk(idx):
            part_ref[idx] = jnp.dot(
                a_ref[pl.ds(idx * CH, CH), :], b_ref[...],
                preferred_element_type=jnp.float32,
            ).astype(jnp.bfloat16)

        compute_chunk(r)
        seed = part_ref[r]
        scw[0] = seed[:, :HN]
        sccw[0] = seed[:, HN:]

        prev_cw = prev_ccw = None
        for h in range(N_DEV - 1):
            slot = h % 2
            r_cw = pltpu.make_async_remote_copy(
                src_ref=scw.at[slot], dst_ref=rcw.at[h],
                send_sem=cw_ssem.at[h], recv_sem=cw_rsem.at[h],
                device_id=(right,), device_id_type=pl.DeviceIdType.MESH,
            )
            r_ccw = pltpu.make_async_remote_copy(
                src_ref=sccw.at[slot], dst_ref=rccw.at[h],
                send_sem=ccw_ssem.at[h], recv_sem=ccw_rsem.at[h],
                device_id=(left,), device_id_type=pl.DeviceIdType.MESH,
            )
            r_cw.start()
            r_ccw.start()
            if prev_cw is not None:
                prev_cw.wait_send()
                prev_ccw.wait_send()
            if h == 0:
                compute_chunk(pidx(-1))
                compute_chunk(pidx(1))
            k = h + 2
            if k <= N_DEV // 2:
                compute_chunk(pidx(-k))
                if k < N_DEV // 2:
                    compute_chunk(pidx(k))
            r_cw.wait_recv()
            r_ccw.wait_recv()
            add_cw = part_ref[pidx(-1 - h)]
            add_ccw = part_ref[pidx(1 + h)]
            acc_cw = rcw[h].astype(jnp.float32) + add_cw[:, :HN].astype(jnp.float32)
            acc_ccw = rccw[h].astype(jnp.float32) + add_ccw[:, HN:].astype(jnp.float32)
            if h < N_DEV - 2:
                scw[1 - slot] = acc_cw.astype(jnp.bfloat16)
                sccw[1 - slot] = acc_ccw.astype(jnp.bfloat16)
            else:
                out_ref[pidx(1), :, pl.ds(0, HN)] = (
                    jnp.maximum(acc_cw, 0.0).astype(jnp.bfloat16))
                out_ref[pidx(-1), :, pl.ds(HN, HN)] = (
                    jnp.maximum(acc_ccw, 0.0).astype(jnp.bfloat16))
            prev_cw, prev_ccw = r_cw, r_ccw
        prev_cw.wait_send()
        prev_ccw.wait_send()

        prev_cw = prev_ccw = None
        for h in range(N_DEV - 1):
            gcw = pidx(1 - h)
            gccw = pidx(-1 + h)
            a_cw = pltpu.make_async_remote_copy(
                src_ref=out_ref.at[gcw, :, pl.ds(0, HN)],
                dst_ref=out_ref.at[gcw, :, pl.ds(0, HN)],
                send_sem=agcw_ssem.at[h], recv_sem=agcw_rsem.at[h],
                device_id=(right,), device_id_type=pl.DeviceIdType.MESH,
            )
            a_ccw = pltpu.make_async_remote_copy(
                src_ref=out_ref.at[gccw, :, pl.ds(HN, HN)],
                dst_ref=out_ref.at[gccw, :, pl.ds(HN, HN)],
                send_sem=agccw_ssem.at[h], recv_sem=agccw_rsem.at[h],
                device_id=(left,), device_id_type=pl.DeviceIdType.MESH,
            )
            a_cw.start()
            a_ccw.start()
            if prev_cw is not None:
                prev_cw.wait_send()
                prev_ccw.wait_send()
            a_cw.wait_recv()
            a_ccw.wait_recv()
            prev_cw, prev_ccw = a_cw, a_ccw
        prev_cw.wait_send()
        prev_ccw.wait_send()

    out3 = pl.pallas_call(
        body,
        out_shape=jax.ShapeDtypeStruct((N_DEV, CH, N), jnp.bfloat16),
        in_specs=[
            pl.BlockSpec(memory_space=pltpu.VMEM),
            pl.BlockSpec(memory_space=pltpu.VMEM),
        ],
        out_specs=pl.BlockSpec(memory_space=pltpu.VMEM),
        scratch_shapes=[
            pltpu.VMEM((N_DEV, CH, N), jnp.bfloat16),
            pltpu.VMEM((N_DEV - 1, CH, HN), jnp.bfloat16),
            pltpu.VMEM((N_DEV - 1, CH, HN), jnp.bfloat16),
            pltpu.VMEM((2, CH, HN), jnp.bfloat16),
            pltpu.VMEM((2, CH, HN), jnp.bfloat16),
            pltpu.SemaphoreType.DMA((N_DEV - 1,)),
            pltpu.SemaphoreType.DMA((N_DEV - 1,)),
            pltpu.SemaphoreType.DMA((N_DEV - 1,)),
            pltpu.SemaphoreType.DMA((N_DEV - 1,)),
            pltpu.SemaphoreType.DMA((N_DEV - 1,)),
            pltpu.SemaphoreType.DMA((N_DEV - 1,)),
            pltpu.SemaphoreType.DMA((N_DEV - 1,)),
            pltpu.SemaphoreType.DMA((N_DEV - 1,)),
        ],
        compiler_params=pltpu.CompilerParams(collective_id=0),
    )(a, b)
    return out3.reshape(M, N)


# device time: 118773 ns/iter; 2.0920x vs baseline; 1.2885x over previous
import jax
import jax.numpy as jnp
from jax import lax
from jax.experimental import pallas as pl
from jax.experimental.pallas import tpu as pltpu

N_DEV = 16
M = 2048
N = 2048
K = 1024
CH = M // N_DEV
NLANE = 4
LW = N // NLANE
HOPS = N_DEV - 1

RING = (0, 4, 8, 12, 13, 14, 15, 11, 10, 9, 5, 6, 7, 3, 2, 1)


def kernel(A, B):
    a = A.astype(jnp.bfloat16)
    b = B.astype(jnp.bfloat16)

    def body(a_ref, b_ref, out_ref, part_ref, *scr):
        recv_bufs = scr[0:NLANE]
        send_bufs = scr[NLANE:2 * NLANE]
        ssems = scr[2 * NLANE:3 * NLANE]
        rsems = scr[3 * NLANE:4 * NLANE]
        agss = scr[4 * NLANE:5 * NLANE]
        agrs = scr[5 * NLANE:6 * NLANE]

        my = lax.axis_index("i")
        zero = my - my
        r = zero
        succ = zero
        pred = zero
        for j in range(N_DEV):
            sel = (my == RING[j]).astype(jnp.int32)
            r = r + j * sel
            succ = succ + RING[(j + 1) % N_DEV] * sel
            pred = pred + RING[(j - 1) % N_DEV] * sel

        def pidx(off):
            return lax.rem(r + 4 * N_DEV + off, N_DEV)

        lane_dst = (succ, succ, pred, pred)
        lane_sgn = (-1, -1, 1, 1)

        barrier_sem = pltpu.get_barrier_semaphore()
        for nbr in (pred, succ):
            pl.semaphore_signal(barrier_sem, inc=1, device_id=(nbr,),
                                device_id_type=pl.DeviceIdType.MESH)
        pl.semaphore_wait(barrier_sem, 2)

        def compute_chunk(idx):
            part_ref[idx] = jnp.dot(
                a_ref[pl.ds(idx * CH, CH), :], b_ref[...],
                preferred_element_type=jnp.float32,
            ).astype(jnp.bfloat16)

        def make_rs(lane, h, slot):
            return pltpu.make_async_remote_copy(
                src_ref=send_bufs[lane].at[slot],
                dst_ref=recv_bufs[lane].at[h],
                send_sem=ssems[lane].at[h],
                recv_sem=rsems[lane].at[h],
                device_id=(lane_dst[lane],),
                device_id_type=pl.DeviceIdType.MESH,
            )

        compute_chunk(r)
        seed = part_ref[r]
        for lane in range(NLANE):
            send_bufs[lane][0] = seed[:, lane * LW:(lane + 1) * LW]

        rs = [[None] * HOPS for _ in range(NLANE)]
        for lane in range(NLANE):
            rs[lane][0] = make_rs(lane, 0, 0)
            rs[lane][0].start()
        compute_chunk(pidx(-1))
        compute_chunk(pidx(1))
        for h in range(HOPS):
            for lane in range(NLANE):
                rs[lane][h].wait_recv()
                add = part_ref[pidx(lane_sgn[lane] * (1 + h))]
                acc = (recv_bufs[lane][h].astype(jnp.float32)
                       + add[:, lane * LW:(lane + 1) * LW].astype(jnp.float32))
                if h < HOPS - 1:
                    slot = (h + 1) % 2
                    if h > 0:
                        rs[lane][h - 1].wait_send()
                    send_bufs[lane][slot] = acc.astype(jnp.bfloat16)
                    rs[lane][h + 1] = make_rs(lane, h + 1, slot)
                    rs[lane][h + 1].start()
                else:
                    own = pidx(-lane_sgn[lane])
                    out_ref[own, :, pl.ds(lane * LW, LW)] = (
                        jnp.maximum(acc, 0.0).astype(jnp.bfloat16))
            k = h + 2
            if k <= N_DEV // 2:
                compute_chunk(pidx(-k))
                if k < N_DEV // 2:
                    compute_chunk(pidx(k))
        for lane in range(NLANE):
            rs[lane][HOPS - 2].wait_send()
            rs[lane][HOPS - 1].wait_send()

        def make_ag(lane, h):
            g = pidx(lane_sgn[lane] * (h - 1))
            reg = out_ref.at[g, :, pl.ds(lane * LW, LW)]
            return pltpu.make_async_remote_copy(
                src_ref=reg, dst_ref=reg,
                send_sem=agss[lane].at[h],
                recv_sem=agrs[lane].at[h],
                device_id=(lane_dst[lane],),
                device_id_type=pl.DeviceIdType.MESH,
            )

        ag = [[None] * HOPS for _ in range(NLANE)]
        for lane in range(NLANE):
            ag[lane][0] = make_ag(lane, 0)
            ag[lane][0].start()
        for h in range(HOPS):
            for lane in range(NLANE):
                ag[lane][h].wait_recv()
                if h < HOPS - 1:
                    if h > 0:
                        ag[lane][h - 1].wait_send()
                    ag[lane][h + 1] = make_ag(lane, h + 1)
                    ag[lane][h + 1].start()
        for lane in range(NLANE):
            ag[lane][HOPS - 2].wait_send()
            ag[lane][HOPS - 1].wait_send()

    scratch = (
        [pltpu.VMEM((HOPS, CH, LW), jnp.bfloat16) for _ in range(NLANE)]
        + [pltpu.VMEM((2, CH, LW), jnp.bfloat16) for _ in range(NLANE)]
        + [pltpu.SemaphoreType.DMA((HOPS,)) for _ in range(4 * NLANE)]
    )
    out3 = pl.pallas_call(
        body,
        out_shape=jax.ShapeDtypeStruct((N_DEV, CH, N), jnp.bfloat16),
        in_specs=[
            pl.BlockSpec(memory_space=pltpu.VMEM),
            pl.BlockSpec(memory_space=pltpu.VMEM),
        ],
        out_specs=pl.BlockSpec(memory_space=pltpu.VMEM),
        scratch_shapes=[pltpu.VMEM((N_DEV, CH, N), jnp.bfloat16)] + scratch,
        compiler_params=pltpu.CompilerParams(collective_id=0),
    )(a, b)
    return out3.reshape(M, N)


# device time: 116790 ns/iter; 2.1275x vs baseline; 1.0170x over previous
import jax
import jax.numpy as jnp
from jax import lax
from jax.experimental import pallas as pl
from jax.experimental.pallas import tpu as pltpu

N_DEV = 16
M = 2048
N = 2048
K = 1024
CH = M // N_DEV
NLANE = 8
LW = N // NLANE
HOPS = N_DEV - 1

RING = (0, 4, 8, 12, 13, 14, 15, 11, 10, 9, 5, 6, 7, 3, 2, 1)


def kernel(A, B):
    a = A.astype(jnp.bfloat16)
    b = B.astype(jnp.bfloat16)

    def body(a_ref, b_ref, out_ref, part_ref, *scr):
        recv_bufs = scr[0:NLANE]
        send_bufs = scr[NLANE:2 * NLANE]
        ssems = scr[2 * NLANE:3 * NLANE]
        rsems = scr[3 * NLANE:4 * NLANE]
        agss = scr[4 * NLANE:5 * NLANE]
        agrs = scr[5 * NLANE:6 * NLANE]

        my = lax.axis_index("i")
        zero = my - my
        r = zero
        succ = zero
        pred = zero
        for j in range(N_DEV):
            sel = (my == RING[j]).astype(jnp.int32)
            r = r + j * sel
            succ = succ + RING[(j + 1) % N_DEV] * sel
            pred = pred + RING[(j - 1) % N_DEV] * sel

        def pidx(off):
            return lax.rem(r + 4 * N_DEV + off, N_DEV)

        lane_dst = tuple(succ if ln < NLANE // 2 else pred
                         for ln in range(NLANE))
        lane_sgn = tuple(-1 if ln < NLANE // 2 else 1
                         for ln in range(NLANE))

        barrier_sem = pltpu.get_barrier_semaphore()
        for nbr in (pred, succ):
            pl.semaphore_signal(barrier_sem, inc=1, device_id=(nbr,),
                                device_id_type=pl.DeviceIdType.MESH)
        pl.semaphore_wait(barrier_sem, 2)

        def compute_chunk(idx):
            part_ref[idx] = jnp.dot(
                a_ref[pl.ds(idx * CH, CH), :], b_ref[...],
                preferred_element_type=jnp.float32,
            ).astype(jnp.bfloat16)

        def make_rs(lane, h, slot):
            return pltpu.make_async_remote_copy(
                src_ref=send_bufs[lane].at[slot],
                dst_ref=recv_bufs[lane].at[h],
                send_sem=ssems[lane].at[h],
                recv_sem=rsems[lane].at[h],
                device_id=(lane_dst[lane],),
                device_id_type=pl.DeviceIdType.MESH,
            )

        compute_chunk(r)
        seed = part_ref[r]
        for lane in range(NLANE):
            send_bufs[lane][0] = seed[:, lane * LW:(lane + 1) * LW]

        rs = [[None] * HOPS for _ in range(NLANE)]
        for lane in range(NLANE):
            rs[lane][0] = make_rs(lane, 0, 0)
            rs[lane][0].start()
        compute_chunk(pidx(-1))
        compute_chunk(pidx(1))
        for h in range(HOPS):
            for lane in range(NLANE):
                rs[lane][h].wait_recv()
                add = part_ref[pidx(lane_sgn[lane] * (1 + h))]
                acc = (recv_bufs[lane][h].astype(jnp.float32)
                       + add[:, lane * LW:(lane + 1) * LW].astype(jnp.float32))
                if h < HOPS - 1:
                    slot = (h + 1) % 2
                    if h > 0:
                        rs[lane][h - 1].wait_send()
                    send_bufs[lane][slot] = acc.astype(jnp.bfloat16)
                    rs[lane][h + 1] = make_rs(lane, h + 1, slot)
                    rs[lane][h + 1].start()
                else:
                    own = pidx(-lane_sgn[lane])
                    out_ref[own, :, pl.ds(lane * LW, LW)] = (
                        jnp.maximum(acc, 0.0).astype(jnp.bfloat16))
            k = h + 2
            if k <= N_DEV // 2:
                compute_chunk(pidx(-k))
                if k < N_DEV // 2:
                    compute_chunk(pidx(k))
        def make_ag(lane, h):
            g = pidx(lane_sgn[lane] * (h - 1))
            reg = out_ref.at[g, :, pl.ds(lane * LW, LW)]
            return pltpu.make_async_remote_copy(
                src_ref=reg, dst_ref=reg,
                send_sem=agss[lane].at[h],
                recv_sem=agrs[lane].at[h],
                device_id=(lane_dst[lane],),
                device_id_type=pl.DeviceIdType.MESH,
            )

        ag = [[None] * HOPS for _ in range(NLANE)]
        for lane in range(NLANE):
            ag[lane][0] = make_ag(lane, 0)
            ag[lane][0].start()
        for lane in range(NLANE):
            rs[lane][HOPS - 2].wait_send()
            rs[lane][HOPS - 1].wait_send()
        for h in range(HOPS):
            for lane in range(NLANE):
                ag[lane][h].wait_recv()
                if h < HOPS - 1:
                    if h > 0:
                        ag[lane][h - 1].wait_send()
                    ag[lane][h + 1] = make_ag(lane, h + 1)
                    ag[lane][h + 1].start()
        for lane in range(NLANE):
            ag[lane][HOPS - 2].wait_send()
            ag[lane][HOPS - 1].wait_send()

    scratch = (
        [pltpu.VMEM((HOPS, CH, LW), jnp.bfloat16) for _ in range(NLANE)]
        + [pltpu.VMEM((2, CH, LW), jnp.bfloat16) for _ in range(NLANE)]
        + [pltpu.SemaphoreType.DMA((HOPS,)) for _ in range(4 * NLANE)]
    )
    out3 = pl.pallas_call(
        body,
        out_shape=jax.ShapeDtypeStruct((N_DEV, CH, N), jnp.bfloat16),
        in_specs=[
            pl.BlockSpec(memory_space=pltpu.VMEM),
            pl.BlockSpec(memory_space=pltpu.VMEM),
        ],
        out_specs=pl.BlockSpec(memory_space=pltpu.VMEM),
        scratch_shapes=[pltpu.VMEM((N_DEV, CH, N), jnp.bfloat16)] + scratch,
        compiler_params=pltpu.CompilerParams(collective_id=0),
    )(a, b)
    return out3.reshape(M, N)


# device time: 107336 ns/iter; 2.3149x vs baseline; 1.0881x over previous
import jax
import jax.numpy as jnp
from jax import lax
from jax.experimental import pallas as pl
from jax.experimental.pallas import tpu as pltpu

N_DEV = 16
M = 2048
N = 2048
CH = 128
NRING = 8
RH = NRING - 1
W0 = 768
W0D = W0 // 2
C1 = W0
W1 = N - W0
W1D = W1 // 2

RING8_X0 = (0, 4, 8, 12, 15, 11, 7, 3)
RING8_X1 = (1, 5, 9, 13, 14, 10, 6, 2)
_PW = {0: 1, 1: 0, 2: 3, 3: 2}
PARTNER = tuple(4 * (m // 4) + _PW[m % 4] for m in range(N_DEV))
XOF = tuple(0 if m % 4 in (0, 3) else 1 for m in range(N_DEV))
_POS = {}
_SUC = {}
_PRE = {}
for ring in (RING8_X0, RING8_X1):
    for p, m in enumerate(ring):
        _POS[m] = p
        _SUC[m] = ring[(p + 1) % NRING]
        _PRE[m] = ring[(p - 1) % NRING]
R8 = tuple(_POS[m] for m in range(N_DEV))
SUC = tuple(_SUC[m] for m in range(N_DEV))
PRE = tuple(_PRE[m] for m in range(N_DEV))
OFF8 = (0, -1, 1, -2, 2, -3, 3, 4)
P04_TRIG = {0: (0,), 1: (1, -1), 4: (4,), 5: (-3, 3), 6: (-2, 2)}
_P04_IDX = {}
_i = 0
for _h in sorted(P04_TRIG):
    for _d in P04_TRIG[_h]:
        _P04_IDX[_d] = _i
        _i += 1

MESHT = pl.DeviceIdType.MESH


def kernel(A, B):
    a = A.astype(jnp.bfloat16)
    b = B.astype(jnp.bfloat16)

    def body(a_ref, b_ref, out_ref, *scr):
        (part, p0recv) = scr[0:2]
        p02r = scr[2:6]
        p02s = scr[6:10]
        p11r = scr[10:14]
        p11s = scr[14:18]
        xacc = scr[18]
        p12recv = scr[19]
        (p0x1_s, p0x1_r) = scr[20:22]
        p02_ss = scr[22:26]
        p02_rs = scr[26:30]
        p03_ss = scr[30:34]
        p03_rs = scr[34:38]
        (p0x4_s, p0x4_r) = scr[38:40]
        p11_ss = scr[40:44]
        p11_rs = scr[44:48]
        (p12_s, p12_r, p13_s, p13_r) = scr[48:52]
        p14_ss = scr[52:56]
        p14_rs = scr[56:60]

        my = lax.axis_index("i")
        zero = my - my
        r8 = zero
        suc = zero
        pre = zero
        partner = zero
        x = zero
        for m in range(N_DEV):
            sel = (my == m).astype(jnp.int32)
            r8 = r8 + R8[m] * sel
            suc = suc + SUC[m] * sel
            pre = pre + PRE[m] * sel
            partner = partner + PARTNER[m] * sel
            x = x + XOF[m] * sel

        def pidx8(off):
            return lax.rem(r8 + 4 * NRING + off, NRING)

        fdir = (0, 0, 1, 1)
        flane = (0, 1, 0, 1)
        ftgt = (suc, suc, pre, pre)
        fsgn = (-1, -1, 1, 1)

        barrier_sem = pltpu.get_barrier_semaphore()
        for nbr in (partner, suc, pre):
            pl.semaphore_signal(barrier_sem, inc=1, device_id=(nbr,),
                                device_id_type=MESHT)
        pl.semaphore_wait(barrier_sem, 3)

        def compute_pair(rc):
            for l in range(2):
                idx = 2 * rc + l
                part[idx] = jnp.dot(
                    a_ref[pl.ds(idx * CH, CH), :], b_ref[...],
                    preferred_element_type=jnp.float32,
                ).astype(jnp.bfloat16)

        def start_p0x1(k):
            rc = pidx8(OFF8[k])
            r = pltpu.make_async_remote_copy(
                src_ref=part.at[2 * rc + (1 - x), :, pl.ds(0, W0)],
                dst_ref=p0recv.at[k],
                send_sem=p0x1_s.at[k], recv_sem=p0x1_r.at[k],
                device_id=(partner,), device_id_type=MESHT)
            r.start()
            return r

        def c1slice(d):
            return pl.ds(C1 + d * W1D, W1D)

        compute_pair(r8)
        for f in range(4):
            p11s[f][0] = part[2 * r8 + flane[f]][:, C1 + fdir[f] * W1D:
                                                 C1 + (fdir[f] + 1) * W1D]

        def make_p11(f, h, slot):
            return pltpu.make_async_remote_copy(
                src_ref=p11s[f].at[slot], dst_ref=p11r[f].at[h],
                send_sem=p11_ss[f].at[h], recv_sem=p11_rs[f].at[h],
                device_id=(ftgt[f],), device_id_type=MESHT)

        rs1 = [[None] * RH for _ in range(4)]
        for f in range(4):
            rs1[f][0] = make_p11(f, 0, 0)
            rs1[f][0].start()
        p0x1 = [None] * NRING
        p0x1[0] = start_p0x1(0)

        p12 = [None] * 2
        for h in range(RH):
            ks = {0: (1, 2), 1: (3, 4), 2: (5, 6), 3: (7,)}.get(h, ())
            for k in ks:
                compute_pair(pidx8(OFF8[k]))
                p0x1[k] = start_p0x1(k)
            for f in range(4):
                rs1[f][h].wait_recv()
                addc = part[2 * pidx8(fsgn[f] * (1 + h)) + flane[f]]
                acc = (p11r[f][h].astype(jnp.float32)
                       + addc[:, C1 + fdir[f] * W1D:
                              C1 + (fdir[f] + 1) * W1D].astype(jnp.float32))
                if h < RH - 1:
                    slot = (h + 1) % 2
                    if h > 0:
                        rs1[f][h - 1].wait_send()
                    p11s[f][slot] = acc.astype(jnp.bfloat16)
                    rs1[f][h + 1] = make_p11(f, h + 1, slot)
                    rs1[f][h + 1].start()
                else:
                    xacc[fdir[f], flane[f]] = acc.astype(jnp.bfloat16)
        for d in range(2):
            p12[d] = pltpu.make_async_remote_copy(
                src_ref=xacc.at[d, 1 - x], dst_ref=p12recv.at[d],
                send_sem=p12_s.at[d], recv_sem=p12_r.at[d],
                device_id=(partner,), device_id_type=MESHT)
            p12[d].start()
        for f in range(4):
            rs1[f][RH - 2].wait_send()
            rs1[f][RH - 1].wait_send()

        for k in range(NRING):
            p0x1[k].wait_recv()
            rc = pidx8(OFF8[k])
            keep = 2 * rc + x
            acc = (p0recv[k].astype(jnp.float32)
                   + part[keep][:, :W0].astype(jnp.float32))
            part[keep, :, pl.ds(0, W0)] = acc.astype(jnp.bfloat16)
        for k in range(NRING):
            p0x1[k].wait_send()

        def make_p02(f, h, slot):
            return pltpu.make_async_remote_copy(
                src_ref=p02s[f].at[slot], dst_ref=p02r[f].at[h],
                send_sem=p02_ss[f].at[h], recv_sem=p02_rs[f].at[h],
                device_id=(ftgt[f],), device_id_type=MESHT)

        seed0 = part[2 * r8 + x]
        for f in range(4):
            p02s[f][0] = seed0[64 * flane[f]:64 * (flane[f] + 1),
                               fdir[f] * W0D:(fdir[f] + 1) * W0D]
        rs0 = [[None] * RH for _ in range(4)]
        for f in range(4):
            rs0[f][0] = make_p02(f, 0, 0)
            rs0[f][0].start()
        for h in range(RH):
            for f in range(4):
                rs0[f][h].wait_recv()
                addc = part[2 * pidx8(fsgn[f] * (1 + h)) + x]
                acc = (p02r[f][h].astype(jnp.float32)
                       + addc[64 * flane[f]:64 * (flane[f] + 1),
                              fdir[f] * W0D:(fdir[f] + 1) * W0D]
                       .astype(jnp.float32))
                if h < RH - 1:
                    slot = (h + 1) % 2
                    if h > 0:
                        rs0[f][h - 1].wait_send()
                    p02s[f][slot] = acc.astype(jnp.bfloat16)
                    rs0[f][h + 1] = make_p02(f, h + 1, slot)
                    rs0[f][h + 1].start()
                else:
                    own = 2 * pidx8(-fsgn[f]) + x
                    out_ref[own, pl.ds(64 * flane[f], 64),
                            pl.ds(fdir[f] * W0D, W0D)] = (
                        jnp.maximum(acc, 0.0).astype(jnp.bfloat16))
        for f in range(4):
            rs0[f][RH - 2].wait_send()
            rs0[f][RH - 1].wait_send()

        p13 = [None] * 2
        for d in range(2):
            p12[d].wait_recv()
            acc = (p12recv[d].astype(jnp.float32)
                   + xacc[d, x].astype(jnp.float32))
            ocd = 2 * pidx8(1 - 2 * d) + x
            out_ref[ocd, :, c1slice(d)] = (
                jnp.maximum(acc, 0.0).astype(jnp.bfloat16))
            reg = out_ref.at[ocd, :, c1slice(d)]
            p13[d] = pltpu.make_async_remote_copy(
                src_ref=reg, dst_ref=reg,
                send_sem=p13_s.at[d], recv_sem=p13_r.at[d],
                device_id=(partner,), device_id_type=MESHT)
            p13[d].start()
            p12[d].wait_send()

        def make_p03(f, h):
            d, l = fdir[f], flane[f]
            g = pidx8((1 - h) if d == 0 else (h - 1))
            reg = out_ref.at[2 * g + x, pl.ds(64 * l, 64),
                             pl.ds(d * W0D, W0D)]
            return pltpu.make_async_remote_copy(
                src_ref=reg, dst_ref=reg,
                send_sem=p03_ss[f].at[h], recv_sem=p03_rs[f].at[h],
                device_id=(ftgt[f],), device_id_type=MESHT)

        def make_p14(f, h):
            d, l = fdir[f], flane[f]
            g = pidx8((1 - h) if d == 0 else (h - 1))
            reg = out_ref.at[2 * g + l, :, c1slice(d)]
            return pltpu.make_async_remote_copy(
                src_ref=reg, dst_ref=reg,
                send_sem=p14_ss[f].at[h], recv_sem=p14_rs[f].at[h],
                device_id=(ftgt[f],), device_id_type=MESHT)

        ag0 = [[None] * RH for _ in range(4)]
        ag1 = [[None] * RH for _ in range(4)]
        for f in range(4):
            ag0[f][0] = make_p03(f, 0)
            ag0[f][0].start()
        for d in range(2):
            p13[d].wait_recv()
            p13[d].wait_send()
        for f in range(4):
            ag1[f][0] = make_p14(f, 0)
            ag1[f][0].start()
        p0x4 = [None] * NRING
        for h in range(RH):
            for f in range(4):
                ag0[f][h].wait_recv()
                if h < RH - 1:
                    if h > 0:
                        ag0[f][h - 1].wait_send()
                    ag0[f][h + 1] = make_p03(f, h + 1)
                    ag0[f][h + 1].start()
            for delta in P04_TRIG.get(h, ()):
                i = _P04_IDX[delta]
                reg = out_ref.at[2 * pidx8(delta) + x, :, pl.ds(0, W0)]
                p0x4[i] = pltpu.make_async_remote_copy(
                    src_ref=reg, dst_ref=reg,
                    send_sem=p0x4_s.at[i], recv_sem=p0x4_r.at[i],
                    device_id=(partner,), device_id_type=MESHT)
                p0x4[i].start()
            for f in range(4):
                ag1[f][h].wait_recv()
                if h < RH - 1:
                    if h > 0:
                        ag1[f][h - 1].wait_send()
                    ag1[f][h + 1] = make_p14(f, h + 1)
                    ag1[f][h + 1].start()
        for f in range(4):
            ag0[f][RH - 2].wait_send()
            ag0[f][RH - 1].wait_send()
            ag1[f][RH - 2].wait_send()
            ag1[f][RH - 1].wait_send()
        for i in range(NRING):
            p0x4[i].wait_recv()
            p0x4[i].wait_send()

    scratch = [
        pltpu.VMEM((N_DEV, CH, N), jnp.bfloat16),
        pltpu.VMEM((NRING, CH, W0), jnp.bfloat16),
    ]
    scratch += [pltpu.VMEM((RH, 64, W0D), jnp.bfloat16) for _ in range(4)]
    scratch += [pltpu.VMEM((2, 64, W0D), jnp.bfloat16) for _ in range(4)]
    scratch += [pltpu.VMEM((RH, CH, W1D), jnp.bfloat16) for _ in range(4)]
    scratch += [pltpu.VMEM((2, CH, W1D), jnp.bfloat16) for _ in range(4)]
    scratch += [
        pltpu.VMEM((2, 2, CH, W1D), jnp.bfloat16),
        pltpu.VMEM((2, CH, W1D), jnp.bfloat16),
        pltpu.SemaphoreType.DMA((NRING,)),
        pltpu.SemaphoreType.DMA((NRING,)),
    ]
    scratch += [pltpu.SemaphoreType.DMA((RH,)) for _ in range(8)]
    scratch += [pltpu.SemaphoreType.DMA((RH,)) for _ in range(8)]
    scratch += [
        pltpu.SemaphoreType.DMA((NRING,)),
        pltpu.SemaphoreType.DMA((NRING,)),
    ]
    scratch += [pltpu.SemaphoreType.DMA((RH,)) for _ in range(8)]
    scratch += [pltpu.SemaphoreType.DMA((2,)) for _ in range(4)]
    scratch += [pltpu.SemaphoreType.DMA((RH,)) for _ in range(8)]
    out3 = pl.pallas_call(
        body,
        out_shape=jax.ShapeDtypeStruct((N_DEV, CH, N), jnp.bfloat16),
        in_specs=[
            pl.BlockSpec(memory_space=pltpu.VMEM),
            pl.BlockSpec(memory_space=pltpu.VMEM),
        ],
        out_specs=pl.BlockSpec(memory_space=pltpu.VMEM),
        scratch_shapes=scratch,
        compiler_params=pltpu.CompilerParams(collective_id=0),
    )(a, b)
    return out3.reshape(M, N)
